# Initial kernel scaffold; baseline (speedup 1.0000x reference)
import jax
import jax.numpy as jnp
from jax.experimental import pallas as pl


def kernel(locations, box_cls, box_regression, centerness, image_sizes):
    raise NotImplementedError("write your pallas kernel here")


# speedup vs baseline: 5.3919x; 5.3919x over previous
import jax
import jax.numpy as jnp
from jax import lax
from jax.experimental import pallas as pl
from jax.experimental.pallas import tpu as pltpu

PRE_NMS_THRESH = 0.01
PRE_NMS_TOP_N = 1000
NMS_THRESH = 0.6
FPN_POST_NMS_TOP_N = 100

PAD = 1024
BLK = 128
NBLK = PAD // BLK


def _nms_body(sizes_ref, data_ref, out_ref):
    imh = sizes_ref[0, 0].astype(jnp.float32)
    imw = sizes_ref[0, 1].astype(jnp.float32)
    px = data_ref[:, 0, :]
    py = data_ref[:, 1, :]
    r0 = data_ref[:, 2, :]
    r1 = data_ref[:, 3, :]
    r2 = data_ref[:, 4, :]
    r3 = data_ref[:, 5, :]
    s = data_ref[:, 6, :]
    lab = data_ref[:, 7, :]

    x1 = jnp.clip(px - r0, 0.0, imw - 1.0)
    y1 = jnp.clip(py - r1, 0.0, imh - 1.0)
    x2 = jnp.clip(px + r2, 0.0, imw - 1.0)
    y2 = jnp.clip(py + r3, 0.0, imh - 1.0)

    ws = x2 - x1 + 1.0
    hs = y2 - y1 + 1.0
    valid = (s > PRE_NMS_THRESH) & (ws >= 0.0) & (hs >= 0.0)
    sc = s * valid.astype(jnp.float32)

    off = lab * 4096.0
    ox1 = x1 + off
    oy1 = y1 + off
    ox2 = x2 + off
    oy2 = y2 + off
    area = (ox2 - ox1 + 1.0) * (oy2 - oy1 + 1.0)

    idxv = lax.broadcasted_iota(jnp.int32, (2, PAD), 1)
    col_iota = lax.broadcasted_iota(jnp.int32, (2, BLK), 1)

    done = []
    for b in range(NBLK):
        lo = b * BLK
        bx1 = ox1[:, lo:lo + BLK]
        by1 = oy1[:, lo:lo + BLK]
        bx2 = ox2[:, lo:lo + BLK]
        by2 = oy2[:, lo:lo + BLK]
        barea = area[:, lo:lo + BLK]
        ltx = jnp.maximum(ox1[:, :, None], bx1[:, None, :])
        lty = jnp.maximum(oy1[:, :, None], by1[:, None, :])
        rbx = jnp.minimum(ox2[:, :, None], bx2[:, None, :])
        rby = jnp.minimum(oy2[:, :, None], by2[:, None, :])
        wx = jnp.maximum(rbx - ltx + 1.0, 0.0)
        wy = jnp.maximum(rby - lty + 1.0, 0.0)
        inter = wx * wy
        union = area[:, :, None] + barea[:, None, :] - inter
        gtf = ((inter / jnp.maximum(union, 1e-6)) > NMS_THRESH
               ).astype(jnp.float32)

        if done:
            keep_prev = jnp.concatenate(done, axis=1)
            sup_ext = jnp.max(gtf[:, :lo, :] * keep_prev[:, :, None], axis=1)
            kbf = 1.0 - sup_ext
        else:
            kbf = jnp.ones((2, BLK), jnp.float32)

        blk_f = gtf[:, lo:lo + BLK, :]
        for i in range(BLK):
            rowm = blk_f[:, i, :] * (col_iota > i).astype(jnp.float32)
            ki = kbf[:, i:i + 1]
            kbf = kbf * (1.0 - rowm * ki)
        done.append(kbf)

    keepf = jnp.concatenate(done, axis=1)
    keep = keepf > 0.5

    is_real = idxv < PRE_NMS_TOP_N
    kv = keep & valid
    q1 = (kv & is_real).astype(jnp.float32)
    q2 = ((~kv) & is_real).astype(jnp.float32)
    tt = (lax.broadcasted_iota(jnp.int32, (PAD, PAD), 0)
          <= lax.broadcasted_iota(jnp.int32, (PAD, PAD), 1)).astype(jnp.float32)
    c1 = jnp.dot(q1, tt, preferred_element_type=jnp.float32)
    c2 = jnp.dot(q2, tt, preferred_element_type=jnp.float32)
    n1 = c1[:, PAD - 1:PAD]
    pos = jnp.where(q1 > 0.0, c1 - 1.0,
                    jnp.where(q2 > 0.0, n1 + c2 - 1.0, 1e9))

    ksc = sc * keepf
    zeros = jnp.zeros((2, PAD), jnp.float32)
    data6 = jnp.stack([x1, y1, x2, y2, ksc, lab, zeros, zeros], axis=2)

    out_cols = lax.broadcasted_iota(jnp.int32, (PAD, BLK), 1).astype(jnp.float32)
    for img in range(2):
        sel = (pos[img][:, None] == out_cols).astype(jnp.float32)
        res = jnp.dot(sel.T, data6[img], precision=lax.Precision.HIGHEST,
                      preferred_element_type=jnp.float32)
        out_ref[img, :, :] = res[:FPN_POST_NMS_TOP_N, :6]


def _nms_call(sizes, data):
    return pl.pallas_call(
        _nms_body,
        out_shape=jax.ShapeDtypeStruct((2, FPN_POST_NMS_TOP_N, 6), jnp.float32),
        in_specs=[
            pl.BlockSpec(memory_space=pltpu.SMEM),
            pl.BlockSpec(memory_space=pltpu.VMEM),
        ],
        out_specs=pl.BlockSpec(memory_space=pltpu.VMEM),
    )(sizes, data)


def kernel(locations, box_cls, box_regression, centerness, image_sizes):
    n, c, h, w = box_cls.shape
    L = h * w
    cls = jax.nn.sigmoid(box_cls).transpose(0, 2, 3, 1).reshape(n, L, c)
    reg = box_regression.transpose(0, 2, 3, 1).reshape(n, L, 4)
    masked = jnp.where(cls > PRE_NMS_THRESH, cls, 0.0).reshape(n, L * c)
    top_scores, top_idx = lax.top_k(masked, PRE_NMS_TOP_N)
    box_loc = top_idx // c
    labels = (top_idx % c + 1).astype(jnp.float32)
    r = jnp.take_along_axis(reg, box_loc[:, :, None], axis=1)
    p = locations[box_loc]
    rows = jnp.concatenate([p, r, top_scores[:, :, None], labels[:, :, None]],
                           axis=2)
    data = jnp.transpose(rows, (0, 2, 1))
    data = jnp.pad(data, ((0, 0), (0, 0), (0, PAD - PRE_NMS_TOP_N)))
    return _nms_call(image_sizes, data)
